# Initial kernel scaffold; baseline (speedup 1.0000x reference)
import jax
import jax.numpy as jnp
from jax.experimental import pallas as pl


def kernel(z, codebook):
    raise NotImplementedError("write your pallas kernel here")


# speedup vs baseline: 3.3386x; 3.3386x over previous
import jax
import jax.numpy as jnp
from jax.experimental import pallas as pl

_CODEBOOK_SIZE = 1024
_EMB = 32
_COMMIT = 0.25
_CH = 2048


def _vq_body(z_ref, cb_ref, zq_ref, idx_ref, loss_ref):
    zb = z_ref[0]
    cb = cb_ref[...]
    scores = jax.lax.dot_general(
        cb, zb, (((1,), (0,)), ((), ())),
        preferred_element_type=jnp.float32)
    rn = jnp.sum(zb * zb, axis=0, keepdims=True)
    cn = jnp.sum(cb * cb, axis=1, keepdims=True)
    dist = (rn - 2.0 * scores) + cn
    idx = jnp.argmin(dist, axis=0)
    onehot = (jax.lax.broadcasted_iota(jnp.int32, (_CODEBOOK_SIZE, _CH), 0)
              == idx[None, :]).astype(jnp.float32)
    q = jax.lax.dot_general(
        cb, onehot, (((0,), (0,)), ((), ())),
        preferred_element_type=jnp.float32)
    zq_ref[0] = zb + (q - zb)
    idx_ref[0, 0] = idx
    diff = q - zb
    part = jnp.sum(diff * diff, keepdims=True)[:, :1]

    @pl.when((pl.program_id(0) == 0) & (pl.program_id(1) == 0))
    def _init():
        loss_ref[...] = jnp.zeros_like(part)

    loss_ref[...] += part


def kernel(z, codebook):
    b, c, f, h, w = z.shape
    n = f * h * w
    nb = n // _CH
    z3 = z.reshape(b, c, n)
    zq3, idx3, loss = pl.pallas_call(
        _vq_body,
        grid=(b, nb),
        in_specs=[
            pl.BlockSpec((1, c, _CH), lambda i, j: (i, 0, j)),
            pl.BlockSpec((_CODEBOOK_SIZE, _EMB), lambda i, j: (0, 0)),
        ],
        out_specs=[
            pl.BlockSpec((1, c, _CH), lambda i, j: (i, 0, j)),
            pl.BlockSpec((1, 1, _CH), lambda i, j: (i * nb + j, 0, 0)),
            pl.BlockSpec((1, 1), lambda i, j: (0, 0)),
        ],
        out_shape=[
            jax.ShapeDtypeStruct((b, c, n), jnp.float32),
            jax.ShapeDtypeStruct((b * nb, 1, _CH), jnp.int32),
            jax.ShapeDtypeStruct((1, 1), jnp.float32),
        ],
    )(z3, codebook)
    zq = zq3.reshape(b, c, f, h, w)
    commit_loss = loss[0, 0] * (_COMMIT / (b * n * c))
    indices = idx3.reshape(-1, 1)
    return (zq, commit_loss, indices)
